# baseline (speedup 1.0000x reference)
import jax
import jax.numpy as jnp
from jax.experimental import pallas as pl
from jax.experimental.pallas import tpu as pltpu

_BN_EPS = 1e-5
_S = 8


def _make_body(B, C, HW):
    nb = B

    def body(x_hbm, wp_ref, bp_ref, vo_ref, o_hbm,
             in_bufs, out_bufs, in_sems, out_sems):
        for k in range(min(_S, nb)):
            pltpu.make_async_copy(
                x_hbm.at[k], in_bufs.at[k], in_sems.at[k]).start()

        for i in range(nb):
            s_ = i % _S
            if i >= _S:
                pltpu.make_async_copy(
                    out_bufs.at[s_], out_bufs.at[s_], out_sems.at[s_]).wait()
            pltpu.make_async_copy(
                in_bufs.at[s_], in_bufs.at[s_], in_sems.at[s_]).wait()

            x = in_bufs[s_]
            p = jnp.dot(wp_ref[...], x,
                        preferred_element_type=jnp.float32) + bp_ref[...]
            sc = jnp.sum(p[2:3, :] * p[0:1, :], axis=1, keepdims=True)
            y = p[1:2, :] * sc
            out_bufs[s_] = x + (vo_ref[:, 0:1] * y + vo_ref[:, 1:2])

            pltpu.make_async_copy(
                out_bufs.at[s_], o_hbm.at[i], out_sems.at[s_]).start()
            j = i + _S
            if j < nb:
                pltpu.make_async_copy(
                    x_hbm.at[j], in_bufs.at[s_], in_sems.at[s_]).start()

        for k in range(max(0, nb - _S), nb):
            s_ = k % _S
            pltpu.make_async_copy(
                out_bufs.at[s_], out_bufs.at[s_], out_sems.at[s_]).wait()

    return body


def kernel(x, g_w, g_b, theta_w, theta_b, phi_w, phi_b,
           W_w, W_b, bn_gamma, bn_beta, bn_mean, bn_var):
    B, C, H, W = x.shape
    HW = H * W
    x_chw = x.reshape(B, C, HW)
    inv_hw = jnp.float32(1.0 / HW)

    f32 = jnp.float32
    wp = jnp.zeros((8, C), f32)
    wp = wp.at[0, :].set(g_w.astype(f32))
    wp = wp.at[1, :].set(theta_w.astype(f32))
    wp = wp.at[2, :].set(phi_w.astype(f32) * inv_hw)
    bp = jnp.zeros((8, 1), f32)
    bp = bp.at[0, 0].set(g_b[0].astype(f32))
    bp = bp.at[1, 0].set(theta_b[0].astype(f32))
    bp = bp.at[2, 0].set(phi_b[0].astype(f32) * inv_hw)

    inv_std = jax.lax.rsqrt(bn_var.astype(f32) + _BN_EPS)
    scale = bn_gamma.astype(f32) * inv_std
    weff = W_w.astype(f32) * scale
    beff = W_b.astype(f32) * scale + bn_beta.astype(f32) - bn_mean.astype(f32) * scale
    vo = jnp.zeros((C, 8), f32)
    vo = vo.at[:, 0].set(weff)
    vo = vo.at[:, 1].set(beff)

    out_chw = pl.pallas_call(
        _make_body(B, C, HW),
        out_shape=jax.ShapeDtypeStruct((B, C, HW), x.dtype),
        grid=(1,),
        in_specs=[
            pl.BlockSpec(memory_space=pl.ANY),
            pl.BlockSpec((8, C), lambda c: (0, 0)),
            pl.BlockSpec((8, 1), lambda c: (0, 0)),
            pl.BlockSpec((C, 8), lambda c: (0, 0)),
        ],
        out_specs=pl.BlockSpec(memory_space=pl.ANY),
        scratch_shapes=[
            pltpu.VMEM((_S, C, HW), f32),
            pltpu.VMEM((_S, C, HW), f32),
            pltpu.SemaphoreType.DMA((_S,)),
            pltpu.SemaphoreType.DMA((_S,)),
        ],
        compiler_params=pltpu.CompilerParams(
            dimension_semantics=("arbitrary",)),
    )(x_chw, wp, bp, vo)

    return out_chw.reshape(B, C, H, W)


# speedup vs baseline: 1.0025x; 1.0025x over previous
import jax
import jax.numpy as jnp
from jax.experimental import pallas as pl
from jax.experimental.pallas import tpu as pltpu
from jax._src.pallas.mosaic.primitives import async_copy as _async_copy

_BN_EPS = 1e-5
_S = 8


def _make_body(B, C, HW):
    nb = B
    half = C // 2

    def _start_in(x_hbm, in_bufs, in_sems, batch, slot):
        _async_copy(x_hbm.at[batch, 0:half], in_bufs.at[slot, 0:half],
                    in_sems.at[slot, 0], priority=0)
        _async_copy(x_hbm.at[batch, half:C], in_bufs.at[slot, half:C],
                    in_sems.at[slot, 1], priority=1)

    def _wait(bufs, sems, slot):
        pltpu.make_async_copy(bufs.at[slot, 0:half], bufs.at[slot, 0:half],
                              sems.at[slot, 0]).wait()
        pltpu.make_async_copy(bufs.at[slot, half:C], bufs.at[slot, half:C],
                              sems.at[slot, 1]).wait()

    def body(x_hbm, wp_ref, bp_ref, vo_ref, o_hbm,
             in_bufs, out_bufs, in_sems, out_sems):
        for k in range(min(_S, nb)):
            _start_in(x_hbm, in_bufs, in_sems, k, k)

        for i in range(nb):
            s_ = i % _S
            if i >= _S:
                _wait(out_bufs, out_sems, s_)
            _wait(in_bufs, in_sems, s_)

            x = in_bufs[s_]
            p = jnp.dot(wp_ref[...], x,
                        preferred_element_type=jnp.float32) + bp_ref[...]
            sc = jnp.sum(p[2:3, :] * p[0:1, :], axis=1, keepdims=True)
            y = p[1:2, :] * sc
            out_bufs[s_] = x + (vo_ref[:, 0:1] * y + vo_ref[:, 1:2])

            _async_copy(out_bufs.at[s_, 0:half], o_hbm.at[i, 0:half],
                        out_sems.at[s_, 0], priority=0)
            _async_copy(out_bufs.at[s_, half:C], o_hbm.at[i, half:C],
                        out_sems.at[s_, 1], priority=1)
            j = i + _S
            if j < nb:
                _start_in(x_hbm, in_bufs, in_sems, j, s_)

        for k in range(max(0, nb - _S), nb):
            _wait(out_bufs, out_sems, k % _S)

    return body


def kernel(x, g_w, g_b, theta_w, theta_b, phi_w, phi_b,
           W_w, W_b, bn_gamma, bn_beta, bn_mean, bn_var):
    B, C, H, W = x.shape
    HW = H * W
    x_chw = x.reshape(B, C, HW)
    inv_hw = jnp.float32(1.0 / HW)

    f32 = jnp.float32
    wp = jnp.zeros((8, C), f32)
    wp = wp.at[0, :].set(g_w.astype(f32))
    wp = wp.at[1, :].set(theta_w.astype(f32))
    wp = wp.at[2, :].set(phi_w.astype(f32) * inv_hw)
    bp = jnp.zeros((8, 1), f32)
    bp = bp.at[0, 0].set(g_b[0].astype(f32))
    bp = bp.at[1, 0].set(theta_b[0].astype(f32))
    bp = bp.at[2, 0].set(phi_b[0].astype(f32) * inv_hw)

    inv_std = jax.lax.rsqrt(bn_var.astype(f32) + _BN_EPS)
    scale = bn_gamma.astype(f32) * inv_std
    weff = W_w.astype(f32) * scale
    beff = W_b.astype(f32) * scale + bn_beta.astype(f32) - bn_mean.astype(f32) * scale
    vo = jnp.zeros((C, 8), f32)
    vo = vo.at[:, 0].set(weff)
    vo = vo.at[:, 1].set(beff)

    out_chw = pl.pallas_call(
        _make_body(B, C, HW),
        out_shape=jax.ShapeDtypeStruct((B, C, HW), x.dtype),
        grid=(1,),
        in_specs=[
            pl.BlockSpec(memory_space=pl.ANY),
            pl.BlockSpec((8, C), lambda c: (0, 0)),
            pl.BlockSpec((8, 1), lambda c: (0, 0)),
            pl.BlockSpec((C, 8), lambda c: (0, 0)),
        ],
        out_specs=pl.BlockSpec(memory_space=pl.ANY),
        scratch_shapes=[
            pltpu.VMEM((_S, C, HW), f32),
            pltpu.VMEM((_S, C, HW), f32),
            pltpu.SemaphoreType.DMA((_S, 2)),
            pltpu.SemaphoreType.DMA((_S, 2)),
        ],
        compiler_params=pltpu.CompilerParams(
            dimension_semantics=("arbitrary",)),
    )(x_chw, wp, bp, vo)

    return out_chw.reshape(B, C, H, W)
